# baseline (speedup 1.0000x reference)
import functools

import jax
import jax.numpy as jnp
from jax import lax
from jax.experimental import pallas as pl
from jax.experimental.pallas import tpu as pltpu
from jax.experimental.pallas import tpu_sc as plsc

N = 10000
D = 128
K1 = 5
K2 = 10

B = 400
GRID = N // B

NC, NS = 2, 16
NW = NC * NS
G2 = N * K1
CG = 40
NCHUNK = G2 // CG
CH_PER_W = (NCHUNK + NW - 1) // NW


def _m2_sc_body(x2_hbm, m2_hbm, buf_in, buf_out):
    wid = lax.axis_index("s") * NC + lax.axis_index("c")

    def chunk_step(t, _):
        chunk = wid + NW * t

        @pl.when(chunk < NCHUNK)
        def _():
            g0 = chunk * CG
            pltpu.sync_copy(x2_hbm.at[pl.ds(g0 * K2, CG * K2)], buf_in)

            def group_step(i, _):
                for d in range(D // 16):
                    sl = pl.ds(16 * d, 16)
                    acc = buf_in[i * K2, sl]
                    for j in range(1, K2):
                        acc = acc + buf_in[i * K2 + j, sl]
                    buf_out[i, sl] = acc * (1.0 / K2)
                return 0

            lax.fori_loop(0, CG, group_step, 0)
            pltpu.sync_copy(buf_out, m2_hbm.at[pl.ds(g0, CG)])

        return 0

    lax.fori_loop(0, CH_PER_W, chunk_step, 0)


_m2_sc = functools.partial(
    pl.kernel,
    out_type=jax.ShapeDtypeStruct((G2, D), jnp.float32),
    mesh=plsc.VectorSubcoreMesh(core_axis_name="c", subcore_axis_name="s"),
    scratch_types=[
        pltpu.VMEM((CG * K2, D), jnp.float32),
        pltpu.VMEM((CG, D), jnp.float32),
    ],
)(_m2_sc_body)


def _tc_body(x0_ref, x1_ref, m2_ref, ws0_ref, wn0_ref, ws1_ref, wn1_ref,
             out_ref):
    f32 = jnp.float32
    ws0 = ws0_ref[...]
    wn0 = wn0_ref[...]

    m1 = None
    mh1 = None
    for j in range(K1):
        x1j = x1_ref[pl.Slice(j, B, K1), :]
        m2j = m2_ref[pl.Slice(j, B, K1), :]
        h1j = jnp.maximum(
            jnp.dot(x1j, ws0, preferred_element_type=f32)
            + jnp.dot(m2j, wn0, preferred_element_type=f32), 0.0)
        m1 = x1j if m1 is None else m1 + x1j
        mh1 = h1j if mh1 is None else mh1 + h1j

    h0 = jnp.maximum(
        jnp.dot(x0_ref[...], ws0, preferred_element_type=f32)
        + jnp.dot(m1 * (1.0 / K1), wn0, preferred_element_type=f32), 0.0)

    out_ref[...] = (
        jnp.dot(h0, ws1_ref[...], preferred_element_type=f32)
        + jnp.dot(mh1 * (1.0 / K1), wn1_ref[...], preferred_element_type=f32))


def kernel(x0, x1, x2, W_self0, W_neigh0, W_self1, W_neigh1):
    m2 = _m2_sc(x2)
    w_spec = pl.BlockSpec((D, D), lambda i: (0, 0))
    return pl.pallas_call(
        _tc_body,
        grid=(GRID,),
        in_specs=[
            pl.BlockSpec((B, D), lambda i: (i, 0)),
            pl.BlockSpec((K1 * B, D), lambda i: (i, 0)),
            pl.BlockSpec((K1 * B, D), lambda i: (i, 0)),
            w_spec, w_spec, w_spec, w_spec,
        ],
        out_specs=pl.BlockSpec((B, D), lambda i: (i, 0)),
        out_shape=jax.ShapeDtypeStruct((N, D), jnp.float32),
        compiler_params=pltpu.CompilerParams(
            dimension_semantics=("parallel",)),
    )(x0, x1, m2, W_self0, W_neigh0, W_self1, W_neigh1)


# speedup vs baseline: 2.9843x; 2.9843x over previous
import jax
import jax.numpy as jnp
from jax.experimental import pallas as pl
from jax.experimental.pallas import tpu as pltpu

N = 10000
D = 128
K1 = 5
K2 = 10

B = 400
GRID = N // B


def _fused_body(x0_ref, x1_ref, x2_ref, ws0_ref, wn0_ref, ws1_ref, wn1_ref,
                out_ref):
    f32 = jnp.float32
    ws0 = ws0_ref[...]
    wn0 = wn0_ref[...]

    m1 = None
    mh1 = None
    for j in range(K1):
        x1j = x1_ref[pl.Slice(j, B, K1), :]
        m2j = x2_ref[pl.Slice(K2 * j, B, K1 * K2), :]
        for u in range(1, K2):
            m2j = m2j + x2_ref[pl.Slice(K2 * j + u, B, K1 * K2), :]
        h1j = jnp.maximum(
            jnp.dot(x1j, ws0, preferred_element_type=f32)
            + jnp.dot(m2j * (1.0 / K2), wn0, preferred_element_type=f32),
            0.0)
        m1 = x1j if m1 is None else m1 + x1j
        mh1 = h1j if mh1 is None else mh1 + h1j

    h0 = jnp.maximum(
        jnp.dot(x0_ref[...], ws0, preferred_element_type=f32)
        + jnp.dot(m1 * (1.0 / K1), wn0, preferred_element_type=f32), 0.0)

    out_ref[...] = (
        jnp.dot(h0, ws1_ref[...], preferred_element_type=f32)
        + jnp.dot(mh1 * (1.0 / K1), wn1_ref[...], preferred_element_type=f32))


def kernel(x0, x1, x2, W_self0, W_neigh0, W_self1, W_neigh1):
    w_spec = pl.BlockSpec((D, D), lambda i: (0, 0))
    return pl.pallas_call(
        _fused_body,
        grid=(GRID,),
        in_specs=[
            pl.BlockSpec((B, D), lambda i: (i, 0)),
            pl.BlockSpec((K1 * B, D), lambda i: (i, 0)),
            pl.BlockSpec((K1 * K2 * B, D), lambda i: (i, 0)),
            w_spec, w_spec, w_spec, w_spec,
        ],
        out_specs=pl.BlockSpec((B, D), lambda i: (i, 0)),
        out_shape=jax.ShapeDtypeStruct((N, D), jnp.float32),
        compiler_params=pltpu.CompilerParams(
            dimension_semantics=("parallel",)),
    )(x0, x1, x2, W_self0, W_neigh0, W_self1, W_neigh1)


# speedup vs baseline: 2.9906x; 1.0021x over previous
import jax
import jax.numpy as jnp
from jax.experimental import pallas as pl
from jax.experimental.pallas import tpu as pltpu

N = 10000
D = 128
K1 = 5
K2 = 10

B = 400
H = B // 2
GRID = N // B


def _half(x0h, x1_ref, x2_ref, ws0, wn0, ws1, wn1, out_ref, base):
    f32 = jnp.float32
    m1 = None
    mh1 = None
    for j in range(K1):
        x1j = x1_ref[pl.Slice(j, H, K1), :]
        m2j = x2_ref[pl.Slice(K2 * j, H, K1 * K2), :]
        for u in range(1, K2):
            m2j = m2j + x2_ref[pl.Slice(K2 * j + u, H, K1 * K2), :]
        h1j = jnp.maximum(
            jnp.dot(x1j, ws0, preferred_element_type=f32)
            + jnp.dot(m2j * (1.0 / K2), wn0, preferred_element_type=f32),
            0.0)
        m1 = x1j if m1 is None else m1 + x1j
        mh1 = h1j if mh1 is None else mh1 + h1j

    h0 = jnp.maximum(
        jnp.dot(x0h, ws0, preferred_element_type=f32)
        + jnp.dot(m1 * (1.0 / K1), wn0, preferred_element_type=f32), 0.0)

    out_ref[pl.Slice(base, H), :] = (
        jnp.dot(h0, ws1, preferred_element_type=f32)
        + jnp.dot(mh1 * (1.0 / K1), wn1, preferred_element_type=f32))


def _fused_body(x0_ref, x1_ref, x2a_ref, x2b_ref, ws0_ref, wn0_ref, ws1_ref,
                wn1_ref, out_ref):
    ws0 = ws0_ref[...]
    wn0 = wn0_ref[...]
    ws1 = ws1_ref[...]
    wn1 = wn1_ref[...]
    _half(x0_ref[pl.Slice(0, H), :], x1_ref.at[pl.Slice(0, K1 * H), :],
          x2a_ref, ws0, wn0, ws1, wn1, out_ref, 0)
    _half(x0_ref[pl.Slice(H, H), :], x1_ref.at[pl.Slice(K1 * H, K1 * H), :],
          x2b_ref, ws0, wn0, ws1, wn1, out_ref, H)


def kernel(x0, x1, x2, W_self0, W_neigh0, W_self1, W_neigh1):
    w_spec = pl.BlockSpec((D, D), lambda i: (0, 0))
    return pl.pallas_call(
        _fused_body,
        grid=(GRID,),
        in_specs=[
            pl.BlockSpec((B, D), lambda i: (i, 0)),
            pl.BlockSpec((K1 * B, D), lambda i: (i, 0)),
            pl.BlockSpec((K1 * K2 * H, D), lambda i: (2 * i, 0)),
            pl.BlockSpec((K1 * K2 * H, D), lambda i: (2 * i + 1, 0)),
            w_spec, w_spec, w_spec, w_spec,
        ],
        out_specs=pl.BlockSpec((B, D), lambda i: (i, 0)),
        out_shape=jax.ShapeDtypeStruct((N, D), jnp.float32),
        compiler_params=pltpu.CompilerParams(
            dimension_semantics=("parallel",)),
    )(x0, x1, x2, x2, W_self0, W_neigh0, W_self1, W_neigh1)
